# baseline (speedup 1.0000x reference)
import jax
import jax.numpy as jnp
from jax.experimental import pallas as pl
from jax.experimental.pallas import tpu as pltpu

B = 128
K = 8
LOCAL_E = 8
H = 2048
I = 1024
RC = 128
NW1 = H // RC
NW2 = I // RC
NS = 3


def _ffn_body(*refs):
    (ids_ref, scl_ref, act_ref, x_ref, smooth_ref, s1_ref, s2_ref) = refs[:7]
    w1 = refs[7:7 + NW1]
    w2 = refs[7 + NW1:7 + NW1 + NW2]
    out_ref = refs[7 + NW1 + NW2]
    h_ref, a_ref = refs[7 + NW1 + NW2 + 1:]

    e = pl.program_id(0)
    s = pl.program_id(1)

    xs = x_ref[...] * smooth_ref[0]

    @pl.when(s == 0)
    def _():
        h_ref[...] = sum(
            jnp.dot(xs[:, q * RC:(q + 1) * RC], w1[q][0],
                    preferred_element_type=jnp.float32) for q in range(NW1 // 2))

    @pl.when(s == 1)
    def _():
        h = s1_ref[0] * (h_ref[...] + sum(
            jnp.dot(xs[:, q * RC:(q + 1) * RC], w1[q][0],
                    preferred_element_type=jnp.float32)
            for q in range(NW1 // 2, NW1)))
        g = h[:, :I]
        u = h[:, I:]
        a_ref[...] = (g * jax.nn.sigmoid(g)) * u

    @pl.when((e == 0) & (s == 0))
    def _():
        out_ref[...] = jnp.zeros_like(out_ref)

    @pl.when(s == 2)
    def _():
        m = (ids_ref[...] == e).astype(jnp.float32)
        w_col = jnp.sum(m * scl_ref[...], axis=1, keepdims=True) * act_ref[...]
        a = a_ref[...]
        part = sum(
            jnp.dot(a[:, j * RC:(j + 1) * RC], w2[j][0],
                    preferred_element_type=jnp.float32) for j in range(NW2))
        out_ref[...] += part * s2_ref[0] * w_col


def kernel(x, expert_ids, smooth_scales, expert_scales, x_active_mask,
           gmm1_weight, gmm1_weight_scale, gmm2_weight, gmm2_weight_scale):
    act_col = x_active_mask.astype(jnp.float32).reshape(B, 1)
    smooth3 = smooth_scales.reshape(LOCAL_E, 1, H)
    s1_3 = gmm1_weight_scale.reshape(LOCAL_E, 1, 2 * I)
    s2_3 = gmm2_weight_scale.reshape(LOCAL_E, 1, H)

    grid = (LOCAL_E, NS)

    def w1_spec(q):
        thr = q // (NW1 // 2)

        def imap(e, s, q=q, thr=thr):
            return (jnp.where(s >= thr, e, jnp.maximum(e - 1, 0)), q, 0)

        return pl.BlockSpec((1, RC, 2 * I), imap)

    def w2_spec(j):
        def imap(e, s, j=j):
            return (jnp.where(s >= 2, e, jnp.maximum(e - 1, 0)), j, 0)

        return pl.BlockSpec((1, RC, H), imap)

    out = pl.pallas_call(
        _ffn_body,
        grid=grid,
        in_specs=[
            pl.BlockSpec((B, K), lambda e, s: (0, 0)),
            pl.BlockSpec((B, K), lambda e, s: (0, 0)),
            pl.BlockSpec((B, 1), lambda e, s: (0, 0)),
            pl.BlockSpec((B, H), lambda e, s: (0, 0)),
            pl.BlockSpec((1, 1, H), lambda e, s: (e, 0, 0)),
            pl.BlockSpec((1, 1, 2 * I), lambda e, s: (e, 0, 0)),
            pl.BlockSpec((1, 1, H), lambda e, s: (e, 0, 0)),
        ] + [w1_spec(q) for q in range(NW1)] + [w2_spec(j) for j in range(NW2)],
        out_specs=pl.BlockSpec((B, H), lambda e, s: (0, 0)),
        out_shape=jax.ShapeDtypeStruct((B, H), jnp.float32),
        scratch_shapes=[
            pltpu.VMEM((B, 2 * I), jnp.float32),
            pltpu.VMEM((B, I), jnp.float32),
        ],
        compiler_params=pltpu.CompilerParams(
            dimension_semantics=("arbitrary", "arbitrary"),
        ),
    )(expert_ids, expert_scales, act_col, x, smooth3, s1_3, s2_3,
      *([gmm1_weight] * NW1), *([gmm2_weight] * NW2))
    return out
